# baseline (speedup 1.0000x reference)
import jax
import jax.numpy as jnp
from jax import lax
from jax.experimental import pallas as pl
from jax.experimental.pallas import tpu as pltpu

BATCH = 4
SEQ = 8192
EMB = 1024


def _body(mask_ref, x_ref, ctx_ref, w_ref,
          idx_vmem, wbuf, w_sem, dma_sems):
    m = mask_ref[...].astype(jnp.int32)
    lengths = jnp.sum(m, axis=1)
    idx = jnp.maximum(lengths - 1, 0)
    idx_vmem[...] = idx

    for b in range(BATCH):
        pltpu.make_async_copy(
            x_ref.at[b, idx_vmem[b]], ctx_ref.at[b], dma_sems.at[b]
        ).start()

    iota = lax.broadcasted_iota(jnp.int32, (BATCH, SEQ), 1)
    wbuf[...] = (iota == idx[:, None]).astype(jnp.float32)
    wout = pltpu.make_async_copy(wbuf, w_ref, w_sem)
    wout.start()

    for b in range(BATCH):
        pltpu.make_async_copy(
            x_ref.at[b, idx_vmem[b]], ctx_ref.at[b], dma_sems.at[b]
        ).wait()
    wout.wait()


@jax.jit
def _last_pool(x, padding_mask):
    return pl.pallas_call(
        _body,
        grid=(1,),
        in_specs=[
            pl.BlockSpec((BATCH, SEQ), lambda i: (0, 0)),
            pl.BlockSpec(memory_space=pl.ANY),
        ],
        out_specs=[
            pl.BlockSpec(memory_space=pl.ANY),
            pl.BlockSpec(memory_space=pl.ANY),
        ],
        out_shape=[
            jax.ShapeDtypeStruct((BATCH, EMB), jnp.float32),
            jax.ShapeDtypeStruct((BATCH, SEQ), jnp.float32),
        ],
        scratch_shapes=[
            pltpu.VMEM((BATCH,), jnp.int32),
            pltpu.VMEM((BATCH, SEQ), jnp.float32),
            pltpu.SemaphoreType.DMA,
            pltpu.SemaphoreType.DMA((BATCH,)),
        ],
    )(padding_mask, x)


def kernel(x, padding_mask):
    ctx, w = _last_pool(x, padding_mask)
    return (ctx, w)
